# Initial kernel scaffold; baseline (speedup 1.0000x reference)
import functools

import jax
import jax.numpy as jnp
from jax import lax
from jax.experimental import pallas as pl
from jax.experimental.pallas import tpu as pltpu
from jax.experimental.pallas import tpu_sc as plsc

_NC = 2
_NS = 16
_NW = _NC * _NS


@functools.lru_cache(maxsize=None)
def _build(N, D, CH):
    b_per_w = N // _NW
    n_chunks = b_per_w // CH
    mesh = plsc.VectorSubcoreMesh(core_axis_name="c", subcore_axis_name="s")

    @functools.partial(
        pl.kernel,
        mesh=mesh,
        out_type=jax.ShapeDtypeStruct((N, D), jnp.float32),
        scratch_types=[
            pltpu.VMEM((n_chunks, CH), jnp.int32),
            pltpu.VMEM((CH, D), jnp.float32),
            pltpu.SemaphoreType.DMA,
        ],
    )
    def emb(idx_hbm, table_hbm, out_hbm, idx_v, rows_v, sem):
        wid = lax.axis_index("s") * _NC + lax.axis_index("c")
        base = wid * b_per_w
        pltpu.sync_copy(idx_hbm.at[wid], idx_v)

        def step(j, carry):
            pltpu.async_copy(table_hbm.at[idx_v.at[j]], rows_v, sem).wait()
            pltpu.sync_copy(rows_v, out_hbm.at[pl.ds(base + j * CH, CH)])
            return carry

        lax.fori_loop(0, n_chunks, step, 0)

    return emb


def kernel(X, table):
    B, H = X.shape
    V, D = table.shape
    N = B * H
    CH = 128
    idx = X.reshape(_NW, N // (_NW * CH), CH)
    out = _build(N, D, CH)(idx, table)
    return out.reshape(B, H, D)


# speedup vs baseline: 1.6846x; 1.6846x over previous
import functools

import jax
import jax.numpy as jnp
from jax import lax
from jax.experimental import pallas as pl
from jax.experimental.pallas import tpu as pltpu
from jax.experimental.pallas import tpu_sc as plsc

_NC = 2
_NS = 16
_NW = _NC * _NS


@functools.lru_cache(maxsize=None)
def _build(N, D, CH):
    b_per_w = N // _NW
    n_chunks = b_per_w // CH
    mesh = plsc.VectorSubcoreMesh(core_axis_name="c", subcore_axis_name="s")

    @functools.partial(
        pl.kernel,
        mesh=mesh,
        out_type=jax.ShapeDtypeStruct((N, D), jnp.float32),
        compiler_params=pltpu.CompilerParams(use_tc_tiling_on_sc=False),
        scratch_types=[
            pltpu.VMEM((n_chunks, CH), jnp.int32),
            pltpu.VMEM((CH, D), jnp.float32),
            pltpu.SemaphoreType.DMA,
        ],
    )
    def emb(idx_hbm, table_hbm, out_hbm, idx_v, rows_v, sem):
        wid = lax.axis_index("s") * _NC + lax.axis_index("c")
        base = wid * b_per_w
        pltpu.sync_copy(idx_hbm.at[wid], idx_v)

        def step(j, carry):
            pltpu.async_copy(table_hbm.at[idx_v.at[j]], rows_v, sem).wait()
            pltpu.sync_copy(rows_v, out_hbm.at[pl.ds(base + j * CH, CH)])
            return carry

        lax.fori_loop(0, n_chunks, step, 0)

    return emb


def kernel(X, table):
    B, H = X.shape
    V, D = table.shape
    N = B * H
    CH = 128
    idx = X.reshape(_NW, N // (_NW * CH), CH)
    out = _build(N, D, CH)(idx, table)
    return out.reshape(B, H, D)


# speedup vs baseline: 1.8762x; 1.1137x over previous
import functools

import jax
import jax.numpy as jnp
from jax import lax
from jax.experimental import pallas as pl
from jax.experimental.pallas import tpu as pltpu
from jax.experimental.pallas import tpu_sc as plsc

_NC = 2
_NS = 16
_NW = _NC * _NS


@functools.lru_cache(maxsize=None)
def _build(N, D, CH, NBUF):
    b_per_w = N // _NW
    n_chunks = b_per_w // CH
    n_groups = n_chunks // NBUF
    H = NBUF // 2
    assert n_chunks % NBUF == 0 and n_groups >= 2
    mesh = plsc.VectorSubcoreMesh(core_axis_name="c", subcore_axis_name="s")

    @functools.partial(
        pl.kernel,
        mesh=mesh,
        out_type=jax.ShapeDtypeStruct((N, D), jnp.float32),
        compiler_params=pltpu.CompilerParams(use_tc_tiling_on_sc=False),
        scratch_types=[
            pltpu.VMEM((n_chunks, CH), jnp.int32),
            pltpu.VMEM((NBUF, CH, D), jnp.float32),
            pltpu.SemaphoreType.DMA((NBUF,)),
            pltpu.SemaphoreType.DMA((NBUF,)),
        ],
    )
    def emb(idx_hbm, table_hbm, out_hbm, idx_v, rows_v, gsem, wsem):
        wid = lax.axis_index("s") * _NC + lax.axis_index("c")
        base = wid * b_per_w
        pltpu.sync_copy(idx_hbm.at[wid], idx_v)

        def gather(chunk, buf):
            pltpu.async_copy(table_hbm.at[idx_v.at[chunk]], rows_v.at[buf],
                             gsem.at[buf])

        def gather_wait(chunk, buf):
            pltpu.make_async_copy(table_hbm.at[idx_v.at[chunk]],
                                  rows_v.at[buf], gsem.at[buf]).wait()

        def wb(chunk, buf):
            pltpu.async_copy(rows_v.at[buf],
                             out_hbm.at[pl.ds(base + chunk * CH, CH)],
                             wsem.at[buf])

        def wb_wait(buf):
            pltpu.make_async_copy(rows_v.at[buf],
                                  out_hbm.at[pl.ds(base, CH)],
                                  wsem.at[buf]).wait()

        for c in range(H):
            gather(c, c)

        def group(g, carry):
            for b in range(NBUF):
                j = g * NBUF + b
                b2 = (b + H) % NBUF
                if b < H:
                    @pl.when(g > 0)
                    def _():
                        wb_wait(b2)
                    gather(j + H, b2)
                else:
                    wb_wait(b2)

                    @pl.when(g < n_groups - 1)
                    def _():
                        gather(j + H, b2)
                gather_wait(j, b)
                wb(j, b)
            return carry

        lax.fori_loop(0, n_groups, group, 0)

        for c in range(n_chunks - H, n_chunks):
            wb_wait(c % NBUF)

    return emb


def kernel(X, table):
    B, H = X.shape
    V, D = table.shape
    N = B * H
    CH = 128
    idx = X.reshape(_NW, N // (_NW * CH), CH)
    out = _build(N, D, CH, 8)(idx, table)
    return out.reshape(B, H, D)
